# baseline (speedup 1.0000x reference)
import jax
import jax.numpy as jnp
from jax import lax
from jax.experimental import pallas as pl
from jax.experimental.pallas import tpu as pltpu
from jax.experimental.pallas import tpu_sc as plsc

NC = 2
NS = 16
L = 16
NW = NC * NS

NELEM = 16 * 1 * 512 * 512
NCOL = 512
NROW = NELEM // NCOL
ROWS_PER_TILE = NROW // NW
CHUNK = 32
NCHUNK = ROWS_PER_TILE // CHUNK
CVEC = NCOL // L


def _sc_body(a_hbm, b_hbm, out_hbm, a0, a1, b0, b1, acc,
             sa0, sa1, sb0, sb1):
    cid = lax.axis_index("c")
    sid = lax.axis_index("s")
    wid = sid * NC + cid
    base = wid * ROWS_PER_TILE

    abufs = (a0, a1)
    bbufs = (b0, b1)
    sas = (sa0, sa1)
    sbs = (sb0, sb1)

    def start(i):
        slot = i % 2
        off = base + i * CHUNK
        ha = pltpu.async_copy(a_hbm.at[pl.ds(off, CHUNK)], abufs[slot],
                              sas[slot])
        hb = pltpu.async_copy(b_hbm.at[pl.ds(off, CHUNK)], bbufs[slot],
                              sbs[slot])
        return ha, hb

    handles = [None] * NCHUNK
    handles[0] = start(0)

    zf = jnp.zeros((L,), jnp.float32)
    carry = (zf, zf, zf, zf)

    for i in range(NCHUNK):
        if i + 1 < NCHUNK:
            handles[i + 1] = start(i + 1)
        ha, hb = handles[i]
        ha.wait()
        hb.wait()
        aref = abufs[i % 2]
        bref = bbufs[i % 2]

        def row(r, c, aref=aref, bref=bref):
            s1, c1, s2, c2 = c
            for cc in range(CVEC):
                a = aref[r, pl.ds(cc * L, L)]
                b = bref[r, pl.ds(cc * L, L)]
                d = a - b
                sq = d * d
                m1 = b > 0.0
                m2 = a > 0.0
                s1 = s1 + jnp.where(m1, sq, 0.0)
                s2 = s2 + jnp.where(m2, sq, 0.0)
                c1 = c1 + jnp.where(m1, 1.0, 0.0)
                c2 = c2 + jnp.where(m2, 1.0, 0.0)
            return (s1, c1, s2, c2)

        carry = lax.fori_loop(0, CHUNK, row, carry)

    s1, c1, s2, c2 = carry
    acc[0] = s1
    acc[1] = c1
    acc[2] = s2
    acc[3] = c2
    pltpu.sync_copy(acc, out_hbm.at[wid])


def _sc_reduce(a2d, b2d):
    mesh = plsc.VectorSubcoreMesh(core_axis_name="c", subcore_axis_name="s")
    return pl.kernel(
        _sc_body,
        out_type=jax.ShapeDtypeStruct((NW, 4, L), jnp.float32),
        mesh=mesh,
        compiler_params=pltpu.CompilerParams(use_tc_tiling_on_sc=True,
                                             needs_layout_passes=False),
        scratch_types=[
            pltpu.VMEM((CHUNK, NCOL), jnp.float32),
            pltpu.VMEM((CHUNK, NCOL), jnp.float32),
            pltpu.VMEM((CHUNK, NCOL), jnp.float32),
            pltpu.VMEM((CHUNK, NCOL), jnp.float32),
            pltpu.VMEM((4, L), jnp.float32),
            pltpu.SemaphoreType.DMA,
            pltpu.SemaphoreType.DMA,
            pltpu.SemaphoreType.DMA,
            pltpu.SemaphoreType.DMA,
        ],
    )(a2d, b2d)


def _finish_body(p_ref, out_ref):
    p = p_ref[...]
    comp = lax.broadcasted_iota(jnp.int32, p.shape, 1)
    s1 = jnp.sum(jnp.where(comp == 0, p, 0.0))
    n1 = jnp.sum(jnp.where(comp == 1, p, 0.0))
    s2 = jnp.sum(jnp.where(comp == 2, p, 0.0))
    n2 = jnp.sum(jnp.where(comp == 3, p, 0.0))
    l1 = s1 / n1
    l2 = s2 / n2
    loss = l1 + l2
    col = lax.broadcasted_iota(jnp.int32, (1, 128), 1)
    out_ref[...] = jnp.where(
        col == 0, loss, jnp.where(col == 1, l1,
                                  jnp.where(col == 2, l2, 0.0)))


def _finish(partials):
    return pl.pallas_call(
        _finish_body,
        out_shape=jax.ShapeDtypeStruct((1, 128), jnp.float32),
    )(partials)


def kernel(inputs, targets):
    a2d = inputs.reshape(NROW, NCOL)
    b2d = targets.reshape(NROW, NCOL)
    partials = _sc_reduce(a2d, b2d)
    res = _finish(partials)
    return (res[0, 0], res[0, 1], res[0, 2])


# speedup vs baseline: 5.8481x; 5.8481x over previous
import jax
import jax.numpy as jnp
from jax import lax
from jax.experimental import pallas as pl
from jax.experimental.pallas import tpu as pltpu

NCOL = 512
NROW = (16 * 1 * 512 * 512) // NCOL
TCB = 512
TSTEPS = NROW // TCB


def _tc_body(a_ref, b_ref, out_ref):
    step = pl.program_id(0)

    @pl.when(step == 0)
    def _():
        out_ref[...] = jnp.zeros_like(out_ref)

    a = a_ref[...]
    b = b_ref[...]
    d = a - b
    sq = d * d
    m1 = b > 0.0
    m2 = a > 0.0
    s1 = jnp.sum(jnp.where(m1, sq, 0.0), axis=0, keepdims=True)
    c1 = jnp.sum(jnp.where(m1, 1.0, 0.0), axis=0, keepdims=True)
    s2 = jnp.sum(jnp.where(m2, sq, 0.0), axis=0, keepdims=True)
    c2 = jnp.sum(jnp.where(m2, 1.0, 0.0), axis=0, keepdims=True)
    out_ref[0:1] += s1
    out_ref[1:2] += c1
    out_ref[2:3] += s2
    out_ref[3:4] += c2


def _tc_reduce(a2d, b2d):
    return pl.pallas_call(
        _tc_body,
        grid=(TSTEPS,),
        in_specs=[
            pl.BlockSpec((TCB, NCOL), lambda i: (i, 0)),
            pl.BlockSpec((TCB, NCOL), lambda i: (i, 0)),
        ],
        out_specs=pl.BlockSpec((4, NCOL), lambda i: (0, 0)),
        out_shape=jax.ShapeDtypeStruct((4, NCOL), jnp.float32),
        compiler_params=pltpu.CompilerParams(
            dimension_semantics=("arbitrary",)),
    )(a2d, b2d)


def _finish_body(p_ref, out_ref):
    p = p_ref[...]
    comp = lax.broadcasted_iota(jnp.int32, p.shape, 0)
    s1 = jnp.sum(jnp.where(comp == 0, p, 0.0))
    n1 = jnp.sum(jnp.where(comp == 1, p, 0.0))
    s2 = jnp.sum(jnp.where(comp == 2, p, 0.0))
    n2 = jnp.sum(jnp.where(comp == 3, p, 0.0))
    l1 = s1 / n1
    l2 = s2 / n2
    loss = l1 + l2
    col = lax.broadcasted_iota(jnp.int32, (1, 128), 1)
    out_ref[...] = jnp.where(
        col == 0, loss, jnp.where(col == 1, l1,
                                  jnp.where(col == 2, l2, 0.0)))


def _finish(partials):
    return pl.pallas_call(
        _finish_body,
        out_shape=jax.ShapeDtypeStruct((1, 128), jnp.float32),
    )(partials)


def kernel(inputs, targets):
    a2d = inputs.reshape(NROW, NCOL)
    b2d = targets.reshape(NROW, NCOL)
    partials = _tc_reduce(a2d, b2d)
    res = _finish(partials)
    return (res[0, 0], res[0, 1], res[0, 2])
